# baseline (speedup 1.0000x reference)
import functools

import jax
import jax.numpy as jnp
from jax import lax
from jax.experimental import pallas as pl
from jax.experimental.pallas import tpu as pltpu
from jax.experimental.pallas import tpu_sc as plsc

B = 16384
D = 32
H0 = 64
H1 = 32
NROWS = 1000000
NW = 32
BPW = B // NW
CH = 128
NCH = BPW // CH
TC_COLS = 8192


PACK = 128 // D
QROWS = TC_COLS // PACK
NPACKED = pl.cdiv(NROWS, TC_COLS) * QROWS


def _transpose_body(in_ref, out_ref):
    xT = jnp.transpose(in_ref[...])
    out_ref[...] = jnp.concatenate(
        [xT[m * QROWS:(m + 1) * QROWS] for m in range(PACK)], axis=1)


def _transpose_pack(tT):
    grid = pl.cdiv(NROWS, TC_COLS)
    return pl.pallas_call(
        _transpose_body,
        grid=(grid,),
        in_specs=[pl.BlockSpec((D, TC_COLS), lambda i: (0, i))],
        out_specs=pl.BlockSpec((QROWS, 128), lambda i: (i, 0)),
        out_shape=jax.ShapeDtypeStruct((NPACKED, 128), jnp.float32),
    )(tT)


def _gather_sc(user, item, ut128, it128):
    mesh = plsc.VectorSubcoreMesh(core_axis_name="c", subcore_axis_name="s")

    @functools.partial(
        pl.kernel,
        mesh=mesh,
        out_type=(
            jax.ShapeDtypeStruct((B, 128), jnp.float32),
            jax.ShapeDtypeStruct((B, 128), jnp.float32),
        ),
        scratch_types=[
            pltpu.VMEM((NCH, CH), jnp.int32),
            pltpu.VMEM((NCH, CH), jnp.int32),
            pltpu.VMEM((BPW, 128), jnp.float32),
            pltpu.SemaphoreType.DMA,
        ],
    )
    def gather_kernel(u_hbm, i_hbm, ut_hbm, it_hbm, ue_hbm, ie_hbm,
                      uidx, iidx, rows, sem):
        wid = lax.axis_index("s") * 2 + lax.axis_index("c")
        base = wid * BPW
        for j in range(NCH):
            pltpu.sync_copy(u_hbm.at[pl.ds(base + j * CH, CH)], uidx.at[j])
            pltpu.sync_copy(i_hbm.at[pl.ds(base + j * CH, CH)], iidx.at[j])
        for tbl, idx, out in ((ut_hbm, uidx, ue_hbm), (it_hbm, iidx, ie_hbm)):
            copies = [
                pltpu.async_copy(
                    tbl.at[idx.at[j]], rows.at[pl.ds(j * CH, CH)], sem)
                for j in range(NCH)
            ]
            for c in copies:
                c.wait()
            pltpu.sync_copy(rows, out.at[pl.ds(base, BPW)])

    return gather_kernel(user, item, ut128, it128)


def _select32(x128, sel):
    acc = jnp.where(sel == 0, x128[:, 0:D], 0.0)
    for k in range(1, PACK):
        acc += jnp.where(sel == k, x128[:, k * D:(k + 1) * D], 0.0)
    return acc


def _mlp_body(ue_ref, ie_ref, usel_ref, isel_ref, w0u_ref, w0i_ref, b0_ref,
              w1_ref, b1_ref, wo_ref, bo_ref, out_ref):
    ue = _select32(ue_ref[...], usel_ref[...])
    ie = _select32(ie_ref[...], isel_ref[...])
    x0 = jnp.dot(ue, w0u_ref[...], preferred_element_type=jnp.float32)
    x0 += jnp.dot(ie, w0i_ref[...], preferred_element_type=jnp.float32)
    x0 = jnp.maximum(x0 + b0_ref[...], 0.0)
    x1 = jnp.maximum(
        jnp.dot(x0, w1_ref[...], preferred_element_type=jnp.float32)
        + b1_ref[...], 0.0)
    z = jnp.sum(x1 * wo_ref[...], axis=1, keepdims=True) + bo_ref[...]
    out_ref[...] = 1.0 / (1.0 + jnp.exp(-z))


def _mlp_tc(ue, ie, usel, isel, W0u, W0i, b0, W1, b1, wout_row, bout,
            interpret=False):
    Bb = 2048
    return pl.pallas_call(
        _mlp_body,
        grid=(B // Bb,),
        in_specs=[
            pl.BlockSpec((Bb, 128), lambda i: (i, 0)),
            pl.BlockSpec((Bb, 128), lambda i: (i, 0)),
            pl.BlockSpec((Bb, 1), lambda i: (i, 0)),
            pl.BlockSpec((Bb, 1), lambda i: (i, 0)),
            pl.BlockSpec((D, H0), lambda i: (0, 0)),
            pl.BlockSpec((D, H0), lambda i: (0, 0)),
            pl.BlockSpec((1, H0), lambda i: (0, 0)),
            pl.BlockSpec((H0, H1), lambda i: (0, 0)),
            pl.BlockSpec((1, H1), lambda i: (0, 0)),
            pl.BlockSpec((1, H1), lambda i: (0, 0)),
            pl.BlockSpec((1, 1), lambda i: (0, 0)),
        ],
        out_specs=pl.BlockSpec((Bb, 1), lambda i: (i, 0)),
        out_shape=jax.ShapeDtypeStruct((B, 1), jnp.float32),
        interpret=interpret,
    )(ue, ie, usel, isel, W0u, W0i, b0, W1, b1, wout_row, bout)


def kernel(user, item, user_table, item_table, W0, b0, W1, b1, Wout, bout):
    user = user.astype(jnp.int32)
    item = item.astype(jnp.int32)
    ut128 = _transpose_pack(user_table.T)
    it128 = _transpose_pack(item_table.T)
    uq = (user // TC_COLS) * QROWS + (user % QROWS)
    iq = (item // TC_COLS) * QROWS + (item % QROWS)
    ue, ie = _gather_sc(uq, iq, ut128, it128)
    usel = ((user % TC_COLS) // QROWS).reshape(B, 1)
    isel = ((item % TC_COLS) // QROWS).reshape(B, 1)
    return _mlp_tc(ue, ie, usel, isel, W0[:D], W0[D:], b0.reshape(1, H0), W1,
                   b1.reshape(1, H1), Wout.reshape(1, H1),
                   bout.reshape(1, 1))


# speedup vs baseline: 1.6837x; 1.6837x over previous
import functools

import jax
import jax.numpy as jnp
from jax import lax
from jax.experimental import pallas as pl
from jax.experimental.pallas import tpu as pltpu
from jax.experimental.pallas import tpu_sc as plsc

B = 16384
D = 32
H0 = 64
H1 = 32
NROWS = 1000000
NW = 32
BPW = B // NW
CH = 128
NCH = BPW // CH
TC_COLS = 32768


PACK = 128 // D
QROWS = TC_COLS // PACK
NPACKED = pl.cdiv(NROWS, TC_COLS) * QROWS


def _transpose_body(in_ref, out_ref):
    x = in_ref[...]
    lane = lax.broadcasted_iota(jnp.int32, (D, 128), 1)
    row = lax.broadcasted_iota(jnp.int32, (D, 128), 0)
    acc = None
    for m in range(PACK):
        e = jnp.where(lane == m * D + row, 1.0, 0.0)
        xm = x[:, m * QROWS:(m + 1) * QROWS]
        part = lax.dot_general(xm, e, (((0,), (0,)), ((), ())),
                               preferred_element_type=jnp.float32)
        acc = part if acc is None else acc + part
    out_ref[...] = acc


def _transpose_pack(tT):
    grid = pl.cdiv(NROWS, TC_COLS)
    return pl.pallas_call(
        _transpose_body,
        grid=(grid,),
        in_specs=[pl.BlockSpec((D, TC_COLS), lambda i: (0, i))],
        out_specs=pl.BlockSpec((QROWS, 128), lambda i: (i, 0)),
        out_shape=jax.ShapeDtypeStruct((NPACKED, 128), jnp.float32),
        compiler_params=pltpu.CompilerParams(
            fuse_transposed_lhs_in_matmul=True),
    )(tT)


def _gather_sc(idx, tbl128):
    mesh = plsc.VectorSubcoreMesh(core_axis_name="c", subcore_axis_name="s")

    @functools.partial(
        pl.kernel,
        mesh=mesh,
        out_type=jax.ShapeDtypeStruct((B, 128), jnp.float32),
        scratch_types=[
            pltpu.VMEM((NCH, CH), jnp.int32),
            pltpu.VMEM((BPW, 128), jnp.float32),
            pltpu.SemaphoreType.DMA,
        ],
    )
    def gather_kernel(idx_hbm, tbl_hbm, out_hbm, vidx, rows, sem):
        wid = lax.axis_index("s") * 2 + lax.axis_index("c")
        base = wid * BPW
        for j in range(NCH):
            pltpu.sync_copy(idx_hbm.at[pl.ds(base + j * CH, CH)], vidx.at[j])
        copies = [
            pltpu.async_copy(
                tbl_hbm.at[vidx.at[j]], rows.at[pl.ds(j * CH, CH)], sem)
            for j in range(NCH)
        ]
        for c in copies:
            c.wait()
        pltpu.sync_copy(rows, out_hbm.at[pl.ds(base, BPW)])

    return gather_kernel(idx, tbl128)


def _mlp_body(ue_ref, ie_ref, um_ref, im_ref, w0u_ref, w0i_ref, b0_ref,
              w1_ref, b1_ref, wo_ref, bo_ref, out_ref):
    ue = ue_ref[...] * um_ref[...]
    ie = ie_ref[...] * im_ref[...]
    x0 = jnp.dot(ue, w0u_ref[...], preferred_element_type=jnp.float32)
    x0 += jnp.dot(ie, w0i_ref[...], preferred_element_type=jnp.float32)
    x0 = jnp.maximum(x0 + b0_ref[...], 0.0)
    x1 = jnp.maximum(
        jnp.dot(x0, w1_ref[...], preferred_element_type=jnp.float32)
        + b1_ref[...], 0.0)
    z = jnp.sum(x1 * wo_ref[...], axis=1, keepdims=True) + bo_ref[...]
    out_ref[...] = 1.0 / (1.0 + jnp.exp(-z))


def _mlp_tc(ue, ie, umask, imask, W0u, W0i, b0, W1, b1, wout_row, bout,
            interpret=False):
    Bb = 4096
    return pl.pallas_call(
        _mlp_body,
        grid=(B // Bb,),
        in_specs=[
            pl.BlockSpec((Bb, 128), lambda i: (i, 0)),
            pl.BlockSpec((Bb, 128), lambda i: (i, 0)),
            pl.BlockSpec((Bb, 128), lambda i: (i, 0)),
            pl.BlockSpec((Bb, 128), lambda i: (i, 0)),
            pl.BlockSpec((128, H0), lambda i: (0, 0)),
            pl.BlockSpec((128, H0), lambda i: (0, 0)),
            pl.BlockSpec((1, H0), lambda i: (0, 0)),
            pl.BlockSpec((H0, H1), lambda i: (0, 0)),
            pl.BlockSpec((1, H1), lambda i: (0, 0)),
            pl.BlockSpec((1, H1), lambda i: (0, 0)),
            pl.BlockSpec((1, 1), lambda i: (0, 0)),
        ],
        out_specs=pl.BlockSpec((Bb, 1), lambda i: (i, 0)),
        out_shape=jax.ShapeDtypeStruct((B, 1), jnp.float32),
        interpret=interpret,
    )(ue, ie, umask, imask, W0u, W0i, b0, W1, b1, wout_row, bout)


def kernel(user, item, user_table, item_table, W0, b0, W1, b1, Wout, bout):
    user = user.astype(jnp.int32)
    item = item.astype(jnp.int32)
    uq = (user // TC_COLS) * QROWS + (user % QROWS)
    iq = (item // TC_COLS) * QROWS + (item % QROWS)
    ut128 = _transpose_pack(user_table.T)
    ue = _gather_sc(uq, ut128)
    it128 = _transpose_pack(item_table.T)
    ie = _gather_sc(iq, it128)
    usel = ((user % TC_COLS) // QROWS).reshape(B, 1)
    isel = ((item % TC_COLS) // QROWS).reshape(B, 1)
    lane_grp = (jnp.arange(128, dtype=jnp.int32) // D).reshape(1, 128)
    umask = (lane_grp == usel).astype(jnp.float32)
    imask = (lane_grp == isel).astype(jnp.float32)
    W0us = jnp.concatenate([W0[:D]] * PACK, axis=0)
    W0is = jnp.concatenate([W0[D:]] * PACK, axis=0)
    return _mlp_tc(ue, ie, umask, imask, W0us, W0is, b0.reshape(1, H0), W1,
                   b1.reshape(1, H1), Wout.reshape(1, H1),
                   bout.reshape(1, 1))


# speedup vs baseline: 1.6986x; 1.0088x over previous
import functools

import jax
import jax.numpy as jnp
from jax import lax
from jax.experimental import pallas as pl
from jax.experimental.pallas import tpu as pltpu
from jax.experimental.pallas import tpu_sc as plsc

B = 16384
D = 32
H0 = 64
H1 = 32
NROWS = 1000000
NW = 32
BPW = B // NW
CH = 128
NCH = BPW // CH
TC_COLS = 32768


PACK = 128 // D
QROWS = TC_COLS // PACK
NPACKED = pl.cdiv(NROWS, TC_COLS) * QROWS


def _transpose_body(in_ref, out_ref):
    x = in_ref[...]
    lane = lax.broadcasted_iota(jnp.int32, (D, 128), 1)
    row = lax.broadcasted_iota(jnp.int32, (D, 128), 0)
    acc = None
    for m in range(PACK):
        e = jnp.where(lane == m * D + row, 1.0, 0.0)
        xm = x[:, m * QROWS:(m + 1) * QROWS]
        part = lax.dot_general(xm, e, (((0,), (0,)), ((), ())),
                               preferred_element_type=jnp.float32)
        acc = part if acc is None else acc + part
    out_ref[...] = acc


def _transpose_pack(tT):
    grid = pl.cdiv(NROWS, TC_COLS)
    return pl.pallas_call(
        _transpose_body,
        grid=(grid,),
        in_specs=[pl.BlockSpec((D, TC_COLS), lambda i: (0, i))],
        out_specs=pl.BlockSpec((QROWS, 128), lambda i: (i, 0)),
        out_shape=jax.ShapeDtypeStruct((NPACKED, 128), jnp.float32),
        compiler_params=pltpu.CompilerParams(
            fuse_transposed_lhs_in_matmul=True),
    )(tT)


def _gather_sc(idx, tbl128):
    mesh = plsc.VectorSubcoreMesh(core_axis_name="c", subcore_axis_name="s")

    @functools.partial(
        pl.kernel,
        mesh=mesh,
        out_type=jax.ShapeDtypeStruct((B, 128), jnp.float32),
        scratch_types=[
            pltpu.VMEM((NCH, CH), jnp.int32),
            pltpu.VMEM((BPW, 128), jnp.float32),
            pltpu.SemaphoreType.DMA,
        ],
    )
    def gather_kernel(idx_hbm, tbl_hbm, out_hbm, vidx, rows, sem):
        wid = lax.axis_index("s") * 2 + lax.axis_index("c")
        base = wid * BPW
        for j in range(NCH):
            pltpu.sync_copy(idx_hbm.at[pl.ds(base + j * CH, CH)], vidx.at[j])
        copies = [
            pltpu.async_copy(
                tbl_hbm.at[vidx.at[j]], rows.at[pl.ds(j * CH, CH)], sem)
            for j in range(NCH)
        ]
        for c in copies:
            c.wait()
        pltpu.sync_copy(rows, out_hbm.at[pl.ds(base, BPW)])

    return gather_kernel(idx, tbl128)


def _mlp_body(ue_ref, ie_ref, um_ref, im_ref, w0u_ref, w0i_ref, b0_ref,
              w1_ref, b1_ref, wo_ref, bo_ref, out_ref):
    ue = ue_ref[...] * um_ref[...].astype(jnp.float32)
    ie = ie_ref[...] * im_ref[...].astype(jnp.float32)
    x0 = jnp.dot(ue, w0u_ref[...], preferred_element_type=jnp.float32)
    x0 += jnp.dot(ie, w0i_ref[...], preferred_element_type=jnp.float32)
    x0 = jnp.maximum(x0 + b0_ref[...], 0.0)
    x1 = jnp.maximum(
        jnp.dot(x0, w1_ref[...], preferred_element_type=jnp.float32)
        + b1_ref[...], 0.0)
    z = jnp.sum(x1 * wo_ref[...], axis=1, keepdims=True) + bo_ref[...]
    out_ref[...] = 1.0 / (1.0 + jnp.exp(-z))


def _mlp_tc(ue, ie, umask, imask, W0u, W0i, b0, W1, b1, wout_row, bout,
            interpret=False):
    Bb = 8192
    return pl.pallas_call(
        _mlp_body,
        grid=(B // Bb,),
        in_specs=[
            pl.BlockSpec((Bb, 128), lambda i: (i, 0)),
            pl.BlockSpec((Bb, 128), lambda i: (i, 0)),
            pl.BlockSpec((Bb, 128), lambda i: (i, 0)),
            pl.BlockSpec((Bb, 128), lambda i: (i, 0)),
            pl.BlockSpec((128, H0), lambda i: (0, 0)),
            pl.BlockSpec((128, H0), lambda i: (0, 0)),
            pl.BlockSpec((1, H0), lambda i: (0, 0)),
            pl.BlockSpec((H0, H1), lambda i: (0, 0)),
            pl.BlockSpec((1, H1), lambda i: (0, 0)),
            pl.BlockSpec((1, H1), lambda i: (0, 0)),
            pl.BlockSpec((1, 1), lambda i: (0, 0)),
        ],
        out_specs=pl.BlockSpec((Bb, 1), lambda i: (i, 0)),
        out_shape=jax.ShapeDtypeStruct((B, 1), jnp.float32),
        interpret=interpret,
    )(ue, ie, umask, imask, W0u, W0i, b0, W1, b1, wout_row, bout)


def kernel(user, item, user_table, item_table, W0, b0, W1, b1, Wout, bout):
    user = user.astype(jnp.int32)
    item = item.astype(jnp.int32)
    uq = (user // TC_COLS) * QROWS + (user % QROWS)
    iq = (item // TC_COLS) * QROWS + (item % QROWS)
    ut128 = _transpose_pack(user_table.T)
    ue = _gather_sc(uq, ut128)
    it128 = _transpose_pack(item_table.T)
    ie = _gather_sc(iq, it128)
    usel = ((user % TC_COLS) // QROWS).reshape(B, 1)
    isel = ((item % TC_COLS) // QROWS).reshape(B, 1)
    lane_grp = (jnp.arange(128, dtype=jnp.int32) // D).reshape(1, 128)
    umask = (lane_grp == usel).astype(jnp.bfloat16)
    imask = (lane_grp == isel).astype(jnp.bfloat16)
    W0us = jnp.concatenate([W0[:D]] * PACK, axis=0)
    W0is = jnp.concatenate([W0[D:]] * PACK, axis=0)
    return _mlp_tc(ue, ie, umask, imask, W0us, W0is, b0.reshape(1, H0), W1,
                   b1.reshape(1, H1), Wout.reshape(1, H1),
                   bout.reshape(1, 1))
